# Initial kernel scaffold; baseline (speedup 1.0000x reference)
import functools

import jax
import jax.numpy as jnp
from jax import lax
from jax.experimental import pallas as pl
from jax.experimental.pallas import tpu as pltpu
from jax.experimental.pallas import tpu_sc as plsc

D_MODEL = 64
GROUP = 128
CHUNK = 8


@functools.cache
def _build(n_groups: int, num_cores: int, num_subcores: int):
    num_workers = num_cores * num_subcores
    groups_per_worker = n_groups // num_workers
    assert groups_per_worker * num_workers == n_groups
    assert groups_per_worker % CHUNK == 0

    mesh = plsc.VectorSubcoreMesh(core_axis_name="c", subcore_axis_name="s")

    @functools.partial(
        pl.kernel,
        out_type=jax.ShapeDtypeStruct((n_groups, GROUP, D_MODEL), jnp.float32),
        mesh=mesh,
        scratch_types=[
            pltpu.VMEM((CHUNK, GROUP), jnp.int32),
            pltpu.VMEM((CHUNK, GROUP, D_MODEL), jnp.float32),
            pltpu.SemaphoreType.DMA,
        ],
    )
    def gather_kernel(idx_hbm, table_hbm, out_hbm, idx_v, rows_v, sem):
        wid = lax.axis_index("s") * num_cores + lax.axis_index("c")
        g0 = wid * groups_per_worker

        @pl.loop(0, groups_per_worker, step=CHUNK)
        def _chunk(i):
            base = g0 + i
            pltpu.sync_copy(idx_hbm.at[pl.ds(base, CHUNK)], idx_v)
            copies = [
                pltpu.async_copy(table_hbm.at[idx_v.at[j]], rows_v.at[j], sem)
                for j in range(CHUNK)
            ]
            for c in copies:
                c.wait()
            pltpu.sync_copy(rows_v, out_hbm.at[pl.ds(base, CHUNK)])

    return gather_kernel


def kernel(x, table):
    batch, max_n = x.shape
    n = batch * max_n
    idx = x.reshape(n // GROUP, GROUP).astype(jnp.int32)
    info = plsc.get_sparse_core_info()
    f = _build(n // GROUP, info.num_cores, info.num_subcores)
    out = f(idx, table)
    return out.reshape(batch, max_n, D_MODEL)


# speedup vs baseline: 6.0583x; 6.0583x over previous
import functools

import jax
import jax.numpy as jnp
from jax import lax
from jax.experimental import pallas as pl
from jax.experimental.pallas import tpu as pltpu
from jax.experimental.pallas import tpu_sc as plsc

D_MODEL = 64
GROUP = 128
CHUNK = 8


@functools.cache
def _build(n_groups: int, num_cores: int, num_subcores: int):
    num_workers = num_cores * num_subcores
    groups_per_worker = n_groups // num_workers
    assert groups_per_worker * num_workers == n_groups
    assert groups_per_worker % CHUNK == 0

    mesh = plsc.VectorSubcoreMesh(core_axis_name="c", subcore_axis_name="s")

    @functools.partial(
        pl.kernel,
        out_type=jax.ShapeDtypeStruct((n_groups, GROUP, D_MODEL), jnp.float32),
        mesh=mesh,
        scratch_types=[
            pltpu.VMEM((CHUNK, GROUP), jnp.int32),
            pltpu.VMEM((CHUNK, GROUP, D_MODEL), jnp.float32),
            pltpu.SemaphoreType.DMA,
        ],
        compiler_params=pltpu.CompilerParams(use_tc_tiling_on_sc=False),
    )
    def gather_kernel(idx_hbm, table_hbm, out_hbm, idx_v, rows_v, sem):
        wid = lax.axis_index("s") * num_cores + lax.axis_index("c")
        g0 = wid * groups_per_worker

        @pl.loop(0, groups_per_worker, step=CHUNK)
        def _chunk(i):
            base = g0 + i
            pltpu.sync_copy(idx_hbm.at[pl.ds(base, CHUNK)], idx_v)
            copies = [
                pltpu.async_copy(table_hbm.at[idx_v.at[j]], rows_v.at[j], sem)
                for j in range(CHUNK)
            ]
            for c in copies:
                c.wait()
            pltpu.sync_copy(rows_v, out_hbm.at[pl.ds(base, CHUNK)])

    return gather_kernel


def kernel(x, table):
    batch, max_n = x.shape
    n = batch * max_n
    idx = x.reshape(n // GROUP, GROUP).astype(jnp.int32)
    info = plsc.get_sparse_core_info()
    f = _build(n // GROUP, info.num_cores, info.num_subcores)
    out = f(idx, table)
    return out.reshape(batch, max_n, D_MODEL)
